# baseline (speedup 1.0000x reference)
import functools

import jax
import jax.numpy as jnp
from jax import lax
from jax.experimental import pallas as pl
from jax.experimental.pallas import tpu as pltpu
from jax.experimental.pallas import tpu_sc as plsc

N = 10000
E = 320000
NFEATS = 128
NHIDS = 128
NHEADS = 8
DHEAD = 16
ALPHA = 0.2
CC = 0.5

NB_ROWS = 2000
EB = 250
ECH = 125
NSUB = EB // ECH
NC, NS = 2, 16
NW = NC * NS
EPW = E // NW
NBLK = EPW // EB
RPT = N // NS


def _prep4_body(ht_ref, hp_ref, hp2_ref, hc_ref,
                wg_ref, bg_ref, asg_ref, adg_ref, bag_ref,
                wx_ref, bx_ref, asx_ref, adx_ref, bax_ref,
                wht_o, tt_o, whp_o, tp_o, whp2_o, tp2_o, tc_o):
    def one(h_ref, w_ref, b_ref, asrc_ref, adst_ref, ba_ref, wh_o, t_o):
        wh = lax.dot_general(h_ref[...], w_ref[...],
                             (((1,), (1,)), ((), ())),
                             preferred_element_type=jnp.float32) + b_ref[...]
        if wh_o is not None:
            wh_o[...] = wh
        ts = lax.dot_general(wh, asrc_ref[...], (((1,), (0,)), ((), ())),
                             preferred_element_type=jnp.float32)
        td = lax.dot_general(wh, adst_ref[...], (((1,), (0,)), ((), ())),
                             preferred_element_type=jnp.float32) + ba_ref[...]
        t_o[...] = jnp.concatenate([ts, td], axis=1)

    one(ht_ref, wg_ref, bg_ref, asg_ref, adg_ref, bag_ref, wht_o, tt_o)
    one(hp_ref, wx_ref, bx_ref, asx_ref, adx_ref, bax_ref, whp_o, tp_o)
    one(hp2_ref, wx_ref, bx_ref, asx_ref, adx_ref, bax_ref, whp2_o, tp2_o)
    one(hc_ref, wx_ref, bx_ref, asx_ref, adx_ref, bax_ref, None, tc_o)


def _prep4(ht, hp, hp2, hc, wg, bg, asg, adg, bag, wx, bx, asx, adx, bax):
    hspec = pl.BlockSpec((NB_ROWS, NFEATS), lambda i: (i, 0))
    wspec = pl.BlockSpec((NHIDS, NFEATS), lambda i: (0, 0))
    bspec = pl.BlockSpec((1, NHIDS), lambda i: (0, 0))
    aspec = pl.BlockSpec((NHIDS, NHEADS), lambda i: (0, 0))
    baspec = pl.BlockSpec((1, NHEADS), lambda i: (0, 0))
    whspec = pl.BlockSpec((NB_ROWS, NHIDS), lambda i: (i, 0))
    tspec = pl.BlockSpec((NB_ROWS, 2 * NHEADS), lambda i: (i, 0))
    whshape = jax.ShapeDtypeStruct((N, NHIDS), jnp.float32)
    tshape = jax.ShapeDtypeStruct((N, 2 * NHEADS), jnp.float32)
    return pl.pallas_call(
        _prep4_body,
        grid=(N // NB_ROWS,),
        in_specs=[hspec, hspec, hspec, hspec,
                  wspec, bspec, aspec, aspec, baspec,
                  wspec, bspec, aspec, aspec, baspec],
        out_specs=[whspec, tspec, whspec, tspec, whspec, tspec, tspec],
        out_shape=[whshape, tshape, whshape, tshape, whshape, tshape,
                   tshape],
    )(ht, hp, hp2, hc, wg, bg, asg, adg, bag, wx, bx, asx, adx, bax)


def _lane_bcast(vec, lane):
    idx = jnp.full((16, 1), lane, jnp.int32)
    return lax.gather(
        vec, idx,
        lax.GatherDimensionNumbers(offset_dims=(), collapsed_slice_dims=(0,),
                                   start_index_map=(0,)),
        (1,), mode=lax.GatherScatterMode.PROMISE_IN_BOUNDS)


def _sc_body(wh_t, wh_p, wh_p2, tt, tp, tp2, tcur, s0, d0, s1, d1, s2, d2,
             acc_out, den_out,
             ixsA, ixdA, ixsB, ixdB, sA, sB, rows,
             acc_sh, den_sh, sga, sgb, sgr, sxA, sxB):
    c = lax.axis_index("c")
    s = lax.axis_index("s")
    wid = c * NS + s
    zero16 = jnp.zeros((16,), jnp.float32)

    IXA = (ixsA, ixdA, sxA)
    IXB = (ixsB, ixdB, sxB)

    r0 = s * RPT
    layers = (
        (wh_t, tt, tt, s0, d0),
        (wh_p, tp, tcur, s1, d1),
        (wh_p2, tp2, tcur, s2, d2),
    )
    for l, (wh, ts_tab, td_tab, se, de) in enumerate(layers):
        def _zr(i, u):
            rows[i // 8, pl.ds((i % 8) * 16, 16)] = zero16
            return u
        lax.fori_loop(0, 25 * 8, _zr, 0)

        def _zd(i, u):
            sA[i, :] = zero16
            return u
        lax.fori_loop(0, 125, _zd, 0)
        for k in range(25):
            pltpu.sync_copy(rows.at[pl.ds(0, 25)],
                            acc_sh.at[pl.ds(r0 + k * 25, 25)])
        for k in range(5):
            pltpu.sync_copy(sA.at[pl.ds(0, 125)],
                            den_sh.at[pl.ds(r0 + k * 125, 125)])
        plsc.subcore_barrier()

        row_base = wid * (EPW // ECH)

        def _idx_issue(bi, IX):
            pltpu.async_copy(se.at[pl.ds(row_base + bi * NSUB, NSUB)],
                             IX[0], IX[2])
            pltpu.async_copy(de.at[pl.ds(row_base + bi * NSUB, NSUB)],
                             IX[1], IX[2])

        def _idx_drain(IX):
            pltpu.make_async_copy(se.at[pl.ds(row_base, NSUB)],
                                  IX[0], IX[2]).wait()
            pltpu.make_async_copy(de.at[pl.ds(row_base, NSUB)],
                                  IX[1], IX[2]).wait()

        def _block(bi, IX):
            _idx_drain(IX)
            ixs, ixd = IX[0], IX[1]
            cps = []
            for j in range(NSUB):
                cps.append(pltpu.async_copy(
                    ts_tab.at[ixs.at[j]], sA.at[pl.ds(j * ECH, ECH)], sga))
                cps.append(pltpu.async_copy(
                    td_tab.at[ixd.at[j]], sB.at[pl.ds(j * ECH, ECH)], sgb))
                cps.append(pltpu.async_copy(
                    wh.at[ixs.at[j]], rows.at[pl.ds(j * ECH, ECH)], sgr))
            for j in range(NSUB):
                cps[3 * j].wait()
                cps[3 * j + 1].wait()

            def _score(b, v):
                e = sA[b, :] + lax.rev(sB[b, :], dimensions=(0,))
                e = jnp.where(e >= 0, e, ALPHA * e)
                sA[b, :] = jnp.exp(e)
                return v
            lax.fori_loop(0, EB, _score, 0)

            for j in range(NSUB):
                pltpu.sync_copy(sA.at[pl.ds(j * ECH, ECH)],
                                den_sh.at[ixd.at[j]], add=True)
            for j in range(NSUB):
                cps[3 * j + 2].wait()

            def _mul(b, v):
                e8 = sA[b, :]
                for h in range(NHEADS):
                    rows[b, pl.ds(h * DHEAD, DHEAD)] = (
                        rows[b, pl.ds(h * DHEAD, DHEAD)] * _lane_bcast(e8, h))
                return v
            lax.fori_loop(0, EB, _mul, 0)

            for j in range(NSUB):
                pltpu.sync_copy(rows.at[pl.ds(j * ECH, ECH)],
                                acc_sh.at[ixd.at[j]], add=True)
            _idx_issue(jnp.minimum(bi + 2, NBLK - 1), IX)

        _idx_issue(0, IXA)
        _idx_issue(1, IXB)

        def _pair(i, u):
            _block(2 * i, IXA)
            _block(2 * i + 1, IXB)
            return u
        lax.fori_loop(0, NBLK // 2, _pair, 0)

        _idx_drain(IXA)
        _idx_drain(IXB)
        plsc.subcore_barrier()

        r0c = s * 624
        pltpu.sync_copy(acc_sh.at[pl.ds(r0c, 624)],
                        acc_out.at[l, c, pl.ds(r0c, 624)])
        pltpu.sync_copy(den_sh.at[pl.ds(r0c, 624)],
                        den_out.at[l, c, pl.ds(r0c, 624)])

        @pl.when(s == 0)
        def _tail():
            pltpu.sync_copy(acc_sh.at[pl.ds(9984, 16)],
                            acc_out.at[l, c, pl.ds(9984, 16)])
            pltpu.sync_copy(den_sh.at[pl.ds(9984, 16)],
                            den_out.at[l, c, pl.ds(9984, 16)])
        plsc.subcore_barrier()


def _sc_edges(wh_t, wh_p, wh_p2, tt, tp, tp2, tcur, s0, d0, s1, d1, s2, d2):
    mesh = plsc.VectorSubcoreMesh(core_axis_name="c", subcore_axis_name="s")
    fn = pl.kernel(
        _sc_body,
        mesh=mesh,
        out_type=[
            jax.ShapeDtypeStruct((3, NC, N, NHIDS), jnp.float32),
            jax.ShapeDtypeStruct((3, NC, N, 16), jnp.float32),
        ],
        scratch_types=(
            [pltpu.VMEM((NSUB, ECH), jnp.int32)] * 4
            + [pltpu.VMEM((EB, 16), jnp.float32)] * 2
            + [pltpu.VMEM((EB, NHIDS), jnp.float32),
               pltpu.VMEM_SHARED((N, NHIDS), jnp.float32),
               pltpu.VMEM_SHARED((N, 16), jnp.float32)]
            + [pltpu.SemaphoreType.DMA] * 5
        ),
        compiler_params=pltpu.CompilerParams(use_tc_tiling_on_sc=False),
    )
    return fn(wh_t, wh_p, wh_p2, tt, tp, tp2, tcur, s0, d0, s1, d1, s2, d2)


def _finish_body(acc_ref, den_ref, wih_ref, whh_ref, bih_ref, bhh_ref,
                 coef_ref, out_ref):
    outs = []
    for l in range(3):
        an = acc_ref[l, 0] + acc_ref[l, 1]
        dn = den_ref[l, 0, :, :NHEADS] + den_ref[l, 1, :, :NHEADS]
        inv = jnp.where(dn > 0, 1.0 / dn, 0.0)
        parts = [an[:, h * DHEAD:(h + 1) * DHEAD] * inv[:, h:h + 1]
                 for h in range(NHEADS)]
        outs.append(jnp.concatenate(parts, axis=1))
    x, hc, hs = outs
    ccf = coef_ref[0, 0]
    csf = coef_ref[0, 1]
    g = coef_ref[0, 2]
    h = ccf * hc + csf * hs
    gi = lax.dot_general(x, wih_ref[...], (((1,), (1,)), ((), ())),
                         preferred_element_type=jnp.float32) + bih_ref[...]
    gh = lax.dot_general(h, whh_ref[...], (((1,), (1,)), ((), ())),
                         preferred_element_type=jnp.float32) + bhh_ref[...]
    r = jax.nn.sigmoid(gi[:, :NHIDS] + gh[:, :NHIDS])
    z = jax.nn.sigmoid(gi[:, NHIDS:2 * NHIDS] + gh[:, NHIDS:2 * NHIDS])
    nn = jnp.tanh(gi[:, 2 * NHIDS:] + r * gh[:, 2 * NHIDS:])
    out = (1.0 - z) * nn + z * h
    out_ref[...] = g * out + (1.0 - g) * x


def _finish(acc, den, wih, whh, bih, bhh, coef):
    return pl.pallas_call(
        _finish_body,
        grid=(N // NB_ROWS,),
        in_specs=[
            pl.BlockSpec((3, NC, NB_ROWS, NHIDS), lambda i: (0, 0, i, 0)),
            pl.BlockSpec((3, NC, NB_ROWS, 16), lambda i: (0, 0, i, 0)),
            pl.BlockSpec((3 * NHIDS, NHIDS), lambda i: (0, 0)),
            pl.BlockSpec((3 * NHIDS, NHIDS), lambda i: (0, 0)),
            pl.BlockSpec((1, 3 * NHIDS), lambda i: (0, 0)),
            pl.BlockSpec((1, 3 * NHIDS), lambda i: (0, 0)),
            pl.BlockSpec(memory_space=pltpu.SMEM),
        ],
        out_specs=pl.BlockSpec((NB_ROWS, NHIDS), lambda i: (i, 0)),
        out_shape=jax.ShapeDtypeStruct((N, NHIDS), jnp.float32),
    )(acc, den, wih, whh, bih, bhh, coef)


def kernel(h_t, hp_prev, hp_prev2, hp_cur, edge_index_intra,
           edge_index_counter, edge_index_support, W_gat, b_gat, a_gat,
           ba_gat, W_x, b_x, a_x, ba_x, weight_ih, weight_hh, bias_ih,
           bias_hh, t):
    f32 = jnp.float32
    Wg = W_gat.reshape(NHIDS, NFEATS)
    Wx = W_x.reshape(NHIDS, NFEATS)
    bg = b_gat.reshape(1, NHIDS)
    bx = b_x.reshape(1, NHIDS)
    eye = jnp.eye(NHEADS, dtype=f32)

    def amats(a):
        a_src = (a[:, :DHEAD, None] * eye[:, None, :]).reshape(NHIDS, NHEADS)
        a_dst = (a[:, DHEAD:, None] * eye[:, None, :]).reshape(NHIDS, NHEADS)
        return a_src, a_dst[:, ::-1]

    asg, adg = amats(a_gat)
    asx, adx = amats(a_x)
    bag = ba_gat[::-1].reshape(1, NHEADS)
    bax = ba_x[::-1].reshape(1, NHEADS)

    (wh_t, t_t, wh_p, t_p, wh_p2, t_p2, t_c) = _prep4(
        h_t, hp_prev, hp_prev2, hp_cur,
        Wg, bg, asg, adg, bag, Wx, bx, asx, adx, bax)

    s0 = edge_index_intra[0].reshape(E // ECH, ECH)
    d0 = edge_index_intra[1].reshape(E // ECH, ECH)
    s1 = edge_index_counter[0].reshape(E // ECH, ECH)
    d1 = edge_index_counter[1].reshape(E // ECH, ECH)
    s2 = edge_index_support[0].reshape(E // ECH, ECH)
    d2 = edge_index_support[1].reshape(E // ECH, ECH)

    acc, den = _sc_edges(wh_t, wh_p, wh_p2, t_t, t_p, t_p2, t_c,
                         s0, d0, s1, d1, s2, d2)

    tv = jnp.asarray(t)
    ccf = jnp.where(tv > 1, CC, 1.0).astype(f32)
    csf = jnp.where(tv > 1, 1.0 - CC, 0.0).astype(f32)
    g = jnp.where(tv > 0, 1.0, 0.0).astype(f32)
    coef = jnp.stack([ccf, csf, g]).reshape(1, 3)

    return _finish(acc, den, weight_ih, weight_hh,
                   bias_ih.reshape(1, -1), bias_hh.reshape(1, -1), coef)


# speedup vs baseline: 1.2516x; 1.2516x over previous
import functools

import jax
import jax.numpy as jnp
from jax import lax
from jax.experimental import pallas as pl
from jax.experimental.pallas import tpu as pltpu
from jax.experimental.pallas import tpu_sc as plsc

N = 10000
E = 320000
NFEATS = 128
NHIDS = 128
NHEADS = 8
DHEAD = 16
ALPHA = 0.2
CC = 0.5

NB_ROWS = 2000
EB = 200
ECH = 100
NSUB = EB // ECH
NC, NS = 2, 16
NW = NC * NS
EPW = E // NW
NBLK = EPW // EB
RPT = N // NS


def _prep4_body(ht_ref, hp_ref, hp2_ref, hc_ref,
                wg_ref, bg_ref, asg_ref, adg_ref, bag_ref,
                wx_ref, bx_ref, asx_ref, adx_ref, bax_ref,
                wht_o, tt_o, whp_o, tp_o, whp2_o, tp2_o, tc_o):
    def one(h_ref, w_ref, b_ref, asrc_ref, adst_ref, ba_ref, wh_o, t_o):
        wh = lax.dot_general(h_ref[...], w_ref[...],
                             (((1,), (1,)), ((), ())),
                             preferred_element_type=jnp.float32) + b_ref[...]
        if wh_o is not None:
            wh_o[...] = wh
        ts = lax.dot_general(wh, asrc_ref[...], (((1,), (0,)), ((), ())),
                             preferred_element_type=jnp.float32)
        td = lax.dot_general(wh, adst_ref[...], (((1,), (0,)), ((), ())),
                             preferred_element_type=jnp.float32) + ba_ref[...]
        t_o[...] = jnp.concatenate([ts, td], axis=1)

    one(ht_ref, wg_ref, bg_ref, asg_ref, adg_ref, bag_ref, wht_o, tt_o)
    one(hp_ref, wx_ref, bx_ref, asx_ref, adx_ref, bax_ref, whp_o, tp_o)
    one(hp2_ref, wx_ref, bx_ref, asx_ref, adx_ref, bax_ref, whp2_o, tp2_o)
    one(hc_ref, wx_ref, bx_ref, asx_ref, adx_ref, bax_ref, None, tc_o)


def _prep4(ht, hp, hp2, hc, wg, bg, asg, adg, bag, wx, bx, asx, adx, bax):
    hspec = pl.BlockSpec((NB_ROWS, NFEATS), lambda i: (i, 0))
    wspec = pl.BlockSpec((NHIDS, NFEATS), lambda i: (0, 0))
    bspec = pl.BlockSpec((1, NHIDS), lambda i: (0, 0))
    aspec = pl.BlockSpec((NHIDS, NHEADS), lambda i: (0, 0))
    baspec = pl.BlockSpec((1, NHEADS), lambda i: (0, 0))
    whspec = pl.BlockSpec((NB_ROWS, NHIDS), lambda i: (i, 0))
    tspec = pl.BlockSpec((NB_ROWS, 2 * NHEADS), lambda i: (i, 0))
    whshape = jax.ShapeDtypeStruct((N, NHIDS), jnp.float32)
    tshape = jax.ShapeDtypeStruct((N, 2 * NHEADS), jnp.float32)
    return pl.pallas_call(
        _prep4_body,
        grid=(N // NB_ROWS,),
        in_specs=[hspec, hspec, hspec, hspec,
                  wspec, bspec, aspec, aspec, baspec,
                  wspec, bspec, aspec, aspec, baspec],
        out_specs=[whspec, tspec, whspec, tspec, whspec, tspec, tspec],
        out_shape=[whshape, tshape, whshape, tshape, whshape, tshape,
                   tshape],
    )(ht, hp, hp2, hc, wg, bg, asg, adg, bag, wx, bx, asx, adx, bax)


def _lane_bcast(vec, lane):
    idx = jnp.full((16, 1), lane, jnp.int32)
    return lax.gather(
        vec, idx,
        lax.GatherDimensionNumbers(offset_dims=(), collapsed_slice_dims=(0,),
                                   start_index_map=(0,)),
        (1,), mode=lax.GatherScatterMode.PROMISE_IN_BOUNDS)


def _sc_body(wh_t, wh_p, wh_p2, tt, tp, tp2, tcur, s0, d0, s1, d1, s2, d2,
             acc_out, den_out,
             ixsA, ixdA, ixsB, ixdB, sA, sB, exb, rows,
             acc_sh, den_sh, sga, sgb, sgr, sxA, sxB):
    c = lax.axis_index("c")
    s = lax.axis_index("s")
    wid = c * NS + s
    zero16 = jnp.zeros((16,), jnp.float32)

    IXA = (ixsA, ixdA, sxA)
    IXB = (ixsB, ixdB, sxB)

    r0 = s * RPT
    layers = (
        (wh_t, tt, tt, s0, d0),
        (wh_p, tp, tcur, s1, d1),
        (wh_p2, tp2, tcur, s2, d2),
    )
    for l, (wh, ts_tab, td_tab, se, de) in enumerate(layers):
        def _zr(i, u):
            rows[i // 8, pl.ds((i % 8) * 16, 16)] = zero16
            return u
        lax.fori_loop(0, 25 * 8, _zr, 0)

        def _zd(i, u):
            exb[i, :] = zero16
            return u
        lax.fori_loop(0, 125, _zd, 0)
        for k in range(25):
            pltpu.sync_copy(rows.at[pl.ds(0, 25)],
                            acc_sh.at[pl.ds(r0 + k * 25, 25)])
        for k in range(5):
            pltpu.sync_copy(exb.at[pl.ds(0, 125)],
                            den_sh.at[pl.ds(r0 + k * 125, 125)])
        plsc.subcore_barrier()

        row_base = wid * (EPW // ECH)

        def _idx_issue(bi, IX):
            pltpu.async_copy(se.at[pl.ds(row_base + bi * NSUB, NSUB)],
                             IX[0], IX[2])
            pltpu.async_copy(de.at[pl.ds(row_base + bi * NSUB, NSUB)],
                             IX[1], IX[2])

        def _idx_drain(IX):
            pltpu.make_async_copy(se.at[pl.ds(row_base, NSUB)],
                                  IX[0], IX[2]).wait()
            pltpu.make_async_copy(de.at[pl.ds(row_base, NSUB)],
                                  IX[1], IX[2]).wait()

        def _block(bi, IX):
            _idx_drain(IX)
            ixs, ixd = IX[0], IX[1]
            cps = []
            for j in range(NSUB):
                cps.append(pltpu.async_copy(
                    ts_tab.at[ixs.at[j]], sA.at[pl.ds(j * ECH, ECH)], sga))
                cps.append(pltpu.async_copy(
                    td_tab.at[ixd.at[j]], sB.at[pl.ds(j * ECH, ECH)], sgb))
                cps.append(pltpu.async_copy(
                    wh.at[ixs.at[j]], rows.at[pl.ds(j * ECH, ECH)], sgr))
            for j in range(NSUB):
                cps[3 * j].wait()
                cps[3 * j + 1].wait()

            def _score(b, v):
                e = sA[b, :] + lax.rev(sB[b, :], dimensions=(0,))
                e = jnp.where(e >= 0, e, ALPHA * e)
                exb[b, :] = jnp.exp(e)
                return v
            lax.fori_loop(0, EB, _score, 0)

            for j in range(NSUB):
                pltpu.sync_copy(exb.at[pl.ds(j * ECH, ECH)],
                                den_sh.at[ixd.at[j]], add=True)
            for j in range(NSUB):
                cps[3 * j + 2].wait()

            def _mul(b, v):
                e8 = exb[b, :]
                for h in range(NHEADS):
                    rows[b, pl.ds(h * DHEAD, DHEAD)] = (
                        rows[b, pl.ds(h * DHEAD, DHEAD)] * _lane_bcast(e8, h))
                return v
            lax.fori_loop(0, EB, _mul, 0)

            for j in range(NSUB):
                pltpu.sync_copy(rows.at[pl.ds(j * ECH, ECH)],
                                acc_sh.at[ixd.at[j]], add=True)
            _idx_issue(jnp.minimum(bi + 2, NBLK - 1), IX)

        _idx_issue(0, IXA)
        _idx_issue(1, IXB)

        def _pair(i, u):
            _block(2 * i, IXA)
            _block(2 * i + 1, IXB)
            return u
        lax.fori_loop(0, NBLK // 2, _pair, 0)

        _idx_drain(IXA)
        _idx_drain(IXB)
        plsc.subcore_barrier()

        r0c = s * 624
        pltpu.sync_copy(acc_sh.at[pl.ds(r0c, 624)],
                        acc_out.at[l, c, pl.ds(r0c, 624)])
        pltpu.sync_copy(den_sh.at[pl.ds(r0c, 624)],
                        den_out.at[l, c, pl.ds(r0c, 624)])

        @pl.when(s == 0)
        def _tail():
            pltpu.sync_copy(acc_sh.at[pl.ds(9984, 16)],
                            acc_out.at[l, c, pl.ds(9984, 16)])
            pltpu.sync_copy(den_sh.at[pl.ds(9984, 16)],
                            den_out.at[l, c, pl.ds(9984, 16)])
        plsc.subcore_barrier()


def _sc_edges(wh_t, wh_p, wh_p2, tt, tp, tp2, tcur, s0, d0, s1, d1, s2, d2):
    mesh = plsc.VectorSubcoreMesh(core_axis_name="c", subcore_axis_name="s")
    fn = pl.kernel(
        _sc_body,
        mesh=mesh,
        out_type=[
            jax.ShapeDtypeStruct((3, NC, N, NHIDS), jnp.float32),
            jax.ShapeDtypeStruct((3, NC, N, 16), jnp.float32),
        ],
        scratch_types=(
            [pltpu.VMEM((NSUB, ECH), jnp.int32)] * 4
            + [pltpu.VMEM((EB, 16), jnp.float32)] * 3
            + [pltpu.VMEM((EB, NHIDS), jnp.float32),
               pltpu.VMEM_SHARED((N, NHIDS), jnp.float32),
               pltpu.VMEM_SHARED((N, 16), jnp.float32)]
            + [pltpu.SemaphoreType.DMA] * 5
        ),
        compiler_params=pltpu.CompilerParams(use_tc_tiling_on_sc=False),
    )
    return fn(wh_t, wh_p, wh_p2, tt, tp, tp2, tcur, s0, d0, s1, d1, s2, d2)


def _finish_body(acc_ref, den_ref, wih_ref, whh_ref, bih_ref, bhh_ref,
                 coef_ref, out_ref):
    outs = []
    for l in range(3):
        an = acc_ref[l, 0] + acc_ref[l, 1]
        dn = den_ref[l, 0, :, :NHEADS] + den_ref[l, 1, :, :NHEADS]
        inv = jnp.where(dn > 0, 1.0 / dn, 0.0)
        parts = [an[:, h * DHEAD:(h + 1) * DHEAD] * inv[:, h:h + 1]
                 for h in range(NHEADS)]
        outs.append(jnp.concatenate(parts, axis=1))
    x, hc, hs = outs
    ccf = coef_ref[0, 0]
    csf = coef_ref[0, 1]
    g = coef_ref[0, 2]
    h = ccf * hc + csf * hs
    gi = lax.dot_general(x, wih_ref[...], (((1,), (1,)), ((), ())),
                         preferred_element_type=jnp.float32) + bih_ref[...]
    gh = lax.dot_general(h, whh_ref[...], (((1,), (1,)), ((), ())),
                         preferred_element_type=jnp.float32) + bhh_ref[...]
    r = jax.nn.sigmoid(gi[:, :NHIDS] + gh[:, :NHIDS])
    z = jax.nn.sigmoid(gi[:, NHIDS:2 * NHIDS] + gh[:, NHIDS:2 * NHIDS])
    nn = jnp.tanh(gi[:, 2 * NHIDS:] + r * gh[:, 2 * NHIDS:])
    out = (1.0 - z) * nn + z * h
    out_ref[...] = g * out + (1.0 - g) * x


def _finish(acc, den, wih, whh, bih, bhh, coef):
    return pl.pallas_call(
        _finish_body,
        grid=(N // NB_ROWS,),
        in_specs=[
            pl.BlockSpec((3, NC, NB_ROWS, NHIDS), lambda i: (0, 0, i, 0)),
            pl.BlockSpec((3, NC, NB_ROWS, 16), lambda i: (0, 0, i, 0)),
            pl.BlockSpec((3 * NHIDS, NHIDS), lambda i: (0, 0)),
            pl.BlockSpec((3 * NHIDS, NHIDS), lambda i: (0, 0)),
            pl.BlockSpec((1, 3 * NHIDS), lambda i: (0, 0)),
            pl.BlockSpec((1, 3 * NHIDS), lambda i: (0, 0)),
            pl.BlockSpec(memory_space=pltpu.SMEM),
        ],
        out_specs=pl.BlockSpec((NB_ROWS, NHIDS), lambda i: (i, 0)),
        out_shape=jax.ShapeDtypeStruct((N, NHIDS), jnp.float32),
    )(acc, den, wih, whh, bih, bhh, coef)


def kernel(h_t, hp_prev, hp_prev2, hp_cur, edge_index_intra,
           edge_index_counter, edge_index_support, W_gat, b_gat, a_gat,
           ba_gat, W_x, b_x, a_x, ba_x, weight_ih, weight_hh, bias_ih,
           bias_hh, t):
    f32 = jnp.float32
    Wg = W_gat.reshape(NHIDS, NFEATS)
    Wx = W_x.reshape(NHIDS, NFEATS)
    bg = b_gat.reshape(1, NHIDS)
    bx = b_x.reshape(1, NHIDS)
    eye = jnp.eye(NHEADS, dtype=f32)

    def amats(a):
        a_src = (a[:, :DHEAD, None] * eye[:, None, :]).reshape(NHIDS, NHEADS)
        a_dst = (a[:, DHEAD:, None] * eye[:, None, :]).reshape(NHIDS, NHEADS)
        return a_src, a_dst[:, ::-1]

    asg, adg = amats(a_gat)
    asx, adx = amats(a_x)
    bag = ba_gat[::-1].reshape(1, NHEADS)
    bax = ba_x[::-1].reshape(1, NHEADS)

    (wh_t, t_t, wh_p, t_p, wh_p2, t_p2, t_c) = _prep4(
        h_t, hp_prev, hp_prev2, hp_cur,
        Wg, bg, asg, adg, bag, Wx, bx, asx, adx, bax)

    s0 = edge_index_intra[0].reshape(E // ECH, ECH)
    d0 = edge_index_intra[1].reshape(E // ECH, ECH)
    s1 = edge_index_counter[0].reshape(E // ECH, ECH)
    d1 = edge_index_counter[1].reshape(E // ECH, ECH)
    s2 = edge_index_support[0].reshape(E // ECH, ECH)
    d2 = edge_index_support[1].reshape(E // ECH, ECH)

    acc, den = _sc_edges(wh_t, wh_p, wh_p2, t_t, t_p, t_p2, t_c,
                         s0, d0, s1, d1, s2, d2)

    tv = jnp.asarray(t)
    ccf = jnp.where(tv > 1, CC, 1.0).astype(f32)
    csf = jnp.where(tv > 1, 1.0 - CC, 0.0).astype(f32)
    g = jnp.where(tv > 0, 1.0, 0.0).astype(f32)
    coef = jnp.stack([ccf, csf, g]).reshape(1, 3)

    return _finish(acc, den, weight_ih, weight_hh,
                   bias_ih.reshape(1, -1), bias_hh.reshape(1, -1), coef)
